# baseline (speedup 1.0000x reference)
import jax
import jax.numpy as jnp
from jax import lax
from jax.experimental import pallas as pl
from jax.experimental.pallas import tpu as pltpu
from jax.experimental.pallas import tpu_sc as plsc

_VOCAB = 16
_HIDDEN = 128
_BLK = 16384

_NW = 32
_K = 2
_ROWS_PER_W = 200
_NCHUNK = _ROWS_PER_W // _K
_NREP = 64


def _tc_kernel(ids_ref, emb_ref, out_ref):
    ids = ids_ref[...]
    iota = lax.broadcasted_iota(jnp.int32, (1, _VOCAB), 1)
    one_hot = (ids == iota).astype(jnp.float32)
    out_ref[...] = lax.dot_general(
        one_hot, emb_ref[...],
        (((1,), (0,)), ((), ())),
        preferred_element_type=jnp.float32,
    )


def _tc_lookup(ids_col, embedding, n):
    return pl.pallas_call(
        _tc_kernel,
        grid=(n // _BLK,),
        in_specs=[
            pl.BlockSpec((_BLK, 1), lambda i: (i, 0)),
            pl.BlockSpec((_VOCAB, _HIDDEN), lambda i: (0, 0)),
        ],
        out_specs=pl.BlockSpec((_BLK, _HIDDEN), lambda i: (i, 0)),
        out_shape=jax.ShapeDtypeStruct((n, _HIDDEN), jnp.float32),
    )(ids_col, embedding)


def _sc_body(ids_hbm, emb_rep_hbm, out_hbm, idx_v,
             rows0_v, rows1_v, gsem0, gsem1):
    c = lax.axis_index("c")
    s = lax.axis_index("s")
    wid = s * 2 + c
    row0 = wid * _ROWS_PER_W
    pltpu.sync_copy(ids_hbm.at[pl.ds(row0, _ROWS_PER_W)], idx_v)

    def gather(r, buf, sem):
        return [
            pltpu.async_copy(
                emb_rep_hbm.at[idx_v.at[r + j]],
                buf.at[pl.ds(j * 128, 128)],
                sem,
            )
            for j in range(_K)
        ]

    def body(i, carry):
        ra = 2 * i * _K
        rb = ra + _K
        cps_a = gather(ra, rows0_v, gsem0)
        cps_b = gather(rb, rows1_v, gsem1)
        for cp in cps_a:
            cp.wait()
        pltpu.sync_copy(
            rows0_v, out_hbm.at[pl.ds((row0 + ra) * 128, _K * 128)])
        for cp in cps_b:
            cp.wait()
        pltpu.sync_copy(
            rows1_v, out_hbm.at[pl.ds((row0 + rb) * 128, _K * 128)])
        return carry

    lax.fori_loop(0, _NCHUNK // 2, body, 0)


def _sc_lookup(ids_2d, emb_rep, n):
    mesh = plsc.VectorSubcoreMesh(core_axis_name="c", subcore_axis_name="s")
    k = pl.kernel(
        _sc_body,
        mesh=mesh,
        out_type=jax.ShapeDtypeStruct((n, _HIDDEN), jnp.float32),
        scratch_types=[
            pltpu.VMEM((_ROWS_PER_W, 128), jnp.int32),
            pltpu.VMEM((_K * 128, _HIDDEN), jnp.float32),
            pltpu.VMEM((_K * 128, _HIDDEN), jnp.float32),
            pltpu.SemaphoreType.DMA,
            pltpu.SemaphoreType.DMA,
        ],
    )
    return k(ids_2d, emb_rep)


def kernel(input_ids, embedding):
    batch, seq = input_ids.shape
    n = batch * seq
    ids_flat = input_ids.reshape(n).astype(jnp.int32)
    emb_rep = jnp.tile(embedding, (_NREP, 1))
    ids_off = ids_flat + _VOCAB * (
        jnp.arange(n, dtype=jnp.int32) % _NREP)
    out0 = _tc_lookup(ids_flat.reshape(n, 1), embedding, n)
    out1 = _sc_lookup(ids_off.reshape(n // 128, 128), emb_rep, n)
    return (out0.reshape(batch, seq, _HIDDEN),
            out1.reshape(batch, seq, _HIDDEN))


# speedup vs baseline: 1.0512x; 1.0512x over previous
import jax
import jax.numpy as jnp
from jax import lax
from jax.experimental import pallas as pl
from jax.experimental.pallas import tpu as pltpu
from jax.experimental.pallas import tpu_sc as plsc

_VOCAB = 16
_HIDDEN = 128
_BLK = 16384

_NW = 32
_K = 2
_ROWS_PER_W = 200
_NCHUNK = _ROWS_PER_W // _K
_NREP = 64


def _tc_kernel(ids_ref, emb_ref, out_ref):
    ids = ids_ref[...]
    iota = lax.broadcasted_iota(jnp.int32, (1, _VOCAB), 1)
    one_hot = (ids == iota).astype(jnp.float32)
    out_ref[...] = lax.dot_general(
        one_hot, emb_ref[...],
        (((1,), (0,)), ((), ())),
        preferred_element_type=jnp.float32,
    )


def _tc_lookup(ids_col, embedding, n):
    return pl.pallas_call(
        _tc_kernel,
        grid=(n // _BLK,),
        in_specs=[
            pl.BlockSpec((_BLK, 1), lambda i: (i, 0)),
            pl.BlockSpec((_VOCAB, _HIDDEN), lambda i: (0, 0)),
        ],
        out_specs=pl.BlockSpec((_BLK, _HIDDEN), lambda i: (i, 0)),
        out_shape=jax.ShapeDtypeStruct((n, _HIDDEN), jnp.float32),
    )(ids_col, embedding)


def _sc_body(ids_hbm, emb_hbm, out_hbm, idx_v, table_v,
             rows0_v, rows1_v, osem0, osem1):
    c = lax.axis_index("c")
    s = lax.axis_index("s")
    wid = s * 2 + c
    row0 = wid * _ROWS_PER_W
    pltpu.sync_copy(ids_hbm.at[pl.ds(row0, _ROWS_PER_W)], idx_v)
    pltpu.sync_copy(emb_hbm, table_v)

    def assemble(chunk_row, buf):
        def per_group(g, carry):
            for r in range(_K):
                idvec = idx_v[chunk_row + r, pl.ds(g * 16, 16)]
                for k in range(16):
                    tok = idvec[k]
                    for j in range(_HIDDEN // 16):
                        buf[r * 128 + g * 16 + k, pl.ds(j * 16, 16)] = (
                            table_v[tok, pl.ds(j * 16, 16)])
            return carry
        lax.fori_loop(0, 8, per_group, 0)

    def drain(buf, sem):
        pltpu.make_async_copy(buf, out_hbm.at[pl.ds(0, _K * 128)], sem).wait()

    def body(i, carry):
        ra = 2 * i * _K
        rb = ra + _K

        @pl.when(i > 0)
        def _():
            drain(rows0_v, osem0)

        assemble(ra, rows0_v)
        pltpu.async_copy(
            rows0_v, out_hbm.at[pl.ds((row0 + ra) * 128, _K * 128)], osem0)

        @pl.when(i > 0)
        def _():
            drain(rows1_v, osem1)

        assemble(rb, rows1_v)
        pltpu.async_copy(
            rows1_v, out_hbm.at[pl.ds((row0 + rb) * 128, _K * 128)], osem1)
        return carry

    lax.fori_loop(0, _NCHUNK // 2, body, 0)
    drain(rows0_v, osem0)
    drain(rows1_v, osem1)


def _sc_lookup(ids_2d, embedding, n):
    mesh = plsc.VectorSubcoreMesh(core_axis_name="c", subcore_axis_name="s")
    k = pl.kernel(
        _sc_body,
        mesh=mesh,
        out_type=jax.ShapeDtypeStruct((n, _HIDDEN), jnp.float32),
        scratch_types=[
            pltpu.VMEM((_ROWS_PER_W, 128), jnp.int32),
            pltpu.VMEM((_VOCAB, _HIDDEN), jnp.float32),
            pltpu.VMEM((_K * 128, _HIDDEN), jnp.float32),
            pltpu.VMEM((_K * 128, _HIDDEN), jnp.float32),
            pltpu.SemaphoreType.DMA,
            pltpu.SemaphoreType.DMA,
        ],
    )
    return k(ids_2d, embedding)


def kernel(input_ids, embedding):
    batch, seq = input_ids.shape
    n = batch * seq
    ids_flat = input_ids.reshape(n).astype(jnp.int32)
    out0 = _tc_lookup(ids_flat.reshape(n, 1), embedding, n)
    out1 = _sc_lookup(ids_flat.reshape(n // 128, 128), embedding, n)
    return (out0.reshape(batch, seq, _HIDDEN),
            out1.reshape(batch, seq, _HIDDEN))


# speedup vs baseline: 1.1028x; 1.0491x over previous
import jax
import jax.numpy as jnp
from jax import lax
from jax.experimental import pallas as pl
from jax.experimental.pallas import tpu as pltpu
from jax.experimental.pallas import tpu_sc as plsc

_VOCAB = 16
_HIDDEN = 128
_BLK = 16384

_NW = 32
_K = 2
_ROWS_PER_W = 200
_NCHUNK = _ROWS_PER_W // _K
_NREP = 64


def _tc_kernel(ids_ref, emb_ref, out_ref):
    ids = ids_ref[...]
    iota = lax.broadcasted_iota(jnp.int32, (1, _VOCAB), 1)
    one_hot = (ids == iota).astype(jnp.float32)
    out_ref[...] = lax.dot_general(
        one_hot, emb_ref[...],
        (((1,), (0,)), ((), ())),
        preferred_element_type=jnp.float32,
    )


def _tc_lookup(ids_col, embedding, n):
    return pl.pallas_call(
        _tc_kernel,
        grid=(n // _BLK,),
        in_specs=[
            pl.BlockSpec((_BLK, 1), lambda i: (i, 0)),
            pl.BlockSpec((_VOCAB, _HIDDEN), lambda i: (0, 0)),
        ],
        out_specs=pl.BlockSpec((_BLK, _HIDDEN), lambda i: (i, 0)),
        out_shape=jax.ShapeDtypeStruct((n, _HIDDEN), jnp.float32),
    )(ids_col, embedding)


def _sc_body(ids_hbm, emb_hbm, out_hbm, idx_v, table_v,
             rows0_v, rows1_v, osem0, osem1):
    c = lax.axis_index("c")
    s = lax.axis_index("s")
    wid = s * 2 + c
    row0 = wid * _ROWS_PER_W
    pltpu.sync_copy(ids_hbm.at[pl.ds(row0, _ROWS_PER_W)], idx_v)
    pltpu.sync_copy(emb_hbm, table_v)

    def assemble(chunk_row, buf):
        @plsc.parallel_loop(0, 8, unroll=2)
        def per_group(g):
            for r in range(_K):
                idvec = idx_v[chunk_row + r, pl.ds(g * 16, 16)]
                for k in range(16):
                    tok = idvec[k]
                    for j in range(_HIDDEN // 16):
                        buf[r * 128 + g * 16 + k, pl.ds(j * 16, 16)] = (
                            table_v[tok, pl.ds(j * 16, 16)])

    def drain(buf, sem):
        pltpu.make_async_copy(buf, out_hbm.at[pl.ds(0, _K * 128)], sem).wait()

    def body(i, carry):
        ra = 2 * i * _K
        rb = ra + _K

        @pl.when(i > 0)
        def _():
            drain(rows0_v, osem0)

        assemble(ra, rows0_v)
        pltpu.async_copy(
            rows0_v, out_hbm.at[pl.ds((row0 + ra) * 128, _K * 128)], osem0)

        @pl.when(i > 0)
        def _():
            drain(rows1_v, osem1)

        assemble(rb, rows1_v)
        pltpu.async_copy(
            rows1_v, out_hbm.at[pl.ds((row0 + rb) * 128, _K * 128)], osem1)
        return carry

    lax.fori_loop(0, _NCHUNK // 2, body, 0)
    drain(rows0_v, osem0)
    drain(rows1_v, osem1)


def _sc_lookup(ids_2d, embedding, n):
    mesh = plsc.VectorSubcoreMesh(core_axis_name="c", subcore_axis_name="s")
    k = pl.kernel(
        _sc_body,
        mesh=mesh,
        out_type=jax.ShapeDtypeStruct((n, _HIDDEN), jnp.float32),
        scratch_types=[
            pltpu.VMEM((_ROWS_PER_W, 128), jnp.int32),
            pltpu.VMEM((_VOCAB, _HIDDEN), jnp.float32),
            pltpu.VMEM((_K * 128, _HIDDEN), jnp.float32),
            pltpu.VMEM((_K * 128, _HIDDEN), jnp.float32),
            pltpu.SemaphoreType.DMA,
            pltpu.SemaphoreType.DMA,
        ],
    )
    return k(ids_2d, embedding)


def kernel(input_ids, embedding):
    batch, seq = input_ids.shape
    n = batch * seq
    ids_flat = input_ids.reshape(n).astype(jnp.int32)
    out0 = _tc_lookup(ids_flat.reshape(n, 1), embedding, n)
    out1 = _sc_lookup(ids_flat.reshape(n // 128, 128), embedding, n)
    return (out0.reshape(batch, seq, _HIDDEN),
            out1.reshape(batch, seq, _HIDDEN))


# speedup vs baseline: 1.4276x; 1.2945x over previous
import jax
import jax.numpy as jnp
from jax import lax
from jax.experimental import pallas as pl
from jax.experimental.pallas import tpu as pltpu
from jax.experimental.pallas import tpu_sc as plsc

_VOCAB = 16
_HIDDEN = 128
_BLK = 16384

_NW = 32
_K = 2
_ROWS_PER_W = 200
_NCHUNK = _ROWS_PER_W // _K
_NREP = 64


def _tc_kernel(ids_ref, emb_ref, out_ref):
    ids = ids_ref[...]
    iota = lax.broadcasted_iota(jnp.int32, (1, _VOCAB), 1)
    one_hot = (ids == iota).astype(jnp.float32)
    out_ref[...] = lax.dot_general(
        one_hot, emb_ref[...],
        (((1,), (0,)), ((), ())),
        preferred_element_type=jnp.float32,
    )


def _tc_lookup(ids_col, embedding, n):
    return pl.pallas_call(
        _tc_kernel,
        grid=(n // _BLK,),
        in_specs=[
            pl.BlockSpec((_BLK, 1), lambda i: (i, 0)),
            pl.BlockSpec((_VOCAB, _HIDDEN), lambda i: (0, 0)),
        ],
        out_specs=pl.BlockSpec((_BLK, _HIDDEN), lambda i: (i, 0)),
        out_shape=jax.ShapeDtypeStruct((n, _HIDDEN), jnp.float32),
    )(ids_col, embedding)


def _sc_body(ids_hbm, emb_hbm, out_hbm, idx_v, table_a, table_b,
             rows0_v, rows1_v, gsem0, gsem1):
    c = lax.axis_index("c")
    s = lax.axis_index("s")
    wid = s * 2 + c
    row0 = wid * _ROWS_PER_W
    pltpu.sync_copy(ids_hbm.at[pl.ds(row0, _ROWS_PER_W)], idx_v)

    @pl.when(c == 0)
    def _():
        pltpu.sync_copy(emb_hbm, table_a)

    @pl.when(c == 1)
    def _():
        pltpu.sync_copy(emb_hbm, table_b)

    def make_loop(table_v):
        def gather(r, buf, sem):
            return [
                pltpu.async_copy(
                    table_v.at[idx_v.at[r + j]],
                    buf.at[pl.ds(j * 128, 128)],
                    sem,
                )
                for j in range(_K)
            ]

        def body(i, carry):
            ra = 2 * i * _K
            rb = ra + _K
            cps_a = gather(ra, rows0_v, gsem0)
            cps_b = gather(rb, rows1_v, gsem1)
            for cp in cps_a:
                cp.wait()
            pltpu.sync_copy(
                rows0_v, out_hbm.at[pl.ds((row0 + ra) * 128, _K * 128)])
            for cp in cps_b:
                cp.wait()
            pltpu.sync_copy(
                rows1_v, out_hbm.at[pl.ds((row0 + rb) * 128, _K * 128)])
            return carry

        return body

    @pl.when(c == 0)
    def _():
        lax.fori_loop(0, _NCHUNK // 2, make_loop(table_a), 0)

    @pl.when(c == 1)
    def _():
        lax.fori_loop(0, _NCHUNK // 2, make_loop(table_b), 0)


def _sc_lookup(ids_2d, embedding, n):
    mesh = plsc.VectorSubcoreMesh(core_axis_name="c", subcore_axis_name="s")
    k = pl.kernel(
        _sc_body,
        mesh=mesh,
        out_type=jax.ShapeDtypeStruct((n, _HIDDEN), jnp.float32),
        scratch_types=[
            pltpu.VMEM((_ROWS_PER_W, 128), jnp.int32),
            pltpu.VMEM_SHARED((_VOCAB, _HIDDEN), jnp.float32),
            pltpu.VMEM_SHARED((_VOCAB, _HIDDEN), jnp.float32),
            pltpu.VMEM((_K * 128, _HIDDEN), jnp.float32),
            pltpu.VMEM((_K * 128, _HIDDEN), jnp.float32),
            pltpu.SemaphoreType.DMA,
            pltpu.SemaphoreType.DMA,
        ],
    )
    return k(ids_2d, embedding)


def kernel(input_ids, embedding):
    batch, seq = input_ids.shape
    n = batch * seq
    ids_flat = input_ids.reshape(n).astype(jnp.int32)
    out0 = _tc_lookup(ids_flat.reshape(n, 1), embedding, n)
    out1 = _sc_lookup(ids_flat.reshape(n // 128, 128), embedding, n)
    return (out0.reshape(batch, seq, _HIDDEN),
            out1.reshape(batch, seq, _HIDDEN))
